# baseline (speedup 1.0000x reference)
import functools
import math

import jax
import jax.numpy as jnp
from jax import lax
from jax.experimental import pallas as pl
from jax.experimental.pallas import tpu as pltpu
from jax.experimental.pallas import tpu_sc as plsc

N = 100000
E = 1600000
NODE_DIM = 128
NUM_BASIS = 20
CUTOFF = 5.0

_NC = 2
_NS = 16
_NW = _NC * _NS
_BPW = 3128
_GCHUNK = 256
_NGCHUNK = 13
_N_PAD = _BPW * _NW

_ZTOT = 3 * N * NODE_DIM
_ZPW = _ZTOT // _NW
_ZTILE = 16384
_ZSH = _ZTILE * _NS
_ZSTREAM = 240000
_NZ = _ZPW // _ZSTREAM


def _sc_gather_kernel(idx_hbm, table_hbm, out_hbm, zero_hbm,
                      idx_v, buf0, buf1, ztile, zshared, gsem, osem, zsem):
    wid = lax.axis_index("s") * _NC + lax.axis_index("c")
    sid = lax.axis_index("s")
    base = wid * _BPW
    count = jnp.where(wid == _NW - 1, N - (_NW - 1) * _BPW, _BPW)
    pltpu.sync_copy(idx_hbm.at[pl.ds(base, _BPW)], idx_v)

    def zinit(j, carry):
        ztile[pl.ds(j * 16, 16)] = jnp.zeros((16,), jnp.float32)
        return carry

    lax.fori_loop(0, _ZTILE // 16, zinit, 0)
    pltpu.sync_copy(ztile, zshared.at[pl.ds(sid * _ZTILE, _ZTILE)])
    plsc.subcore_barrier()
    zbase = wid * _ZPW
    zhandles = [
        pltpu.async_copy(
            zshared.at[pl.ds(0, _ZSTREAM)],
            zero_hbm.at[pl.ds(zbase + k * _ZSTREAM, _ZSTREAM)],
            zsem,
        )
        for k in range(_NZ)
    ]
    bufs = [buf0, buf1]
    starts = [jnp.minimum(i * _GCHUNK, count - _GCHUNK)
              for i in range(_NGCHUNK)]

    def gather_start(i):
        return pltpu.async_copy(
            table_hbm.at[idx_v.at[pl.ds(starts[i], _GCHUNK)]],
            bufs[i % 2], gsem,
        )

    g = {0: gather_start(0), 1: gather_start(1)}
    outs = {}
    for i in range(_NGCHUNK):
        g[i].wait()
        outs[i] = pltpu.async_copy(
            bufs[i % 2], out_hbm.at[pl.ds(base + starts[i], _GCHUNK)], osem
        )
        outs[i].wait()
        if i + 2 < _NGCHUNK:
            g[i + 2] = gather_start(i + 2)
    for h in zhandles:
        h.wait()


def _sc_gather(atomic_numbers, embed_table):
    idx = jnp.pad(atomic_numbers.astype(jnp.int32), (0, _N_PAD - N))
    mesh = plsc.VectorSubcoreMesh(core_axis_name="c", subcore_axis_name="s")
    kern = functools.partial(
        pl.kernel,
        mesh=mesh,
        compiler_params=pltpu.CompilerParams(needs_layout_passes=False),
        out_type=(
            jax.ShapeDtypeStruct((N, NODE_DIM), jnp.float32),
            jax.ShapeDtypeStruct((_ZTOT,), jnp.float32),
        ),
        scratch_types=[
            pltpu.VMEM((_BPW,), jnp.int32),
            pltpu.VMEM((_GCHUNK, NODE_DIM), jnp.float32),
            pltpu.VMEM((_GCHUNK, NODE_DIM), jnp.float32),
            pltpu.VMEM((_ZTILE,), jnp.float32),
            pltpu.VMEM_SHARED((_ZSH,), jnp.float32),
            pltpu.SemaphoreType.DMA,
            pltpu.SemaphoreType.DMA,
            pltpu.SemaphoreType.DMA,
        ],
    )(_sc_gather_kernel)
    node_invariant, zero_flat = kern(idx, embed_table)
    node_equivariant = zero_flat.reshape(3, N, NODE_DIM).transpose(1, 0, 2)
    return node_invariant, node_equivariant


_BE = 12800
_GRID = E // _BE

_PI = math.pi
_A = math.sqrt(2.0 / CUTOFF)

_S3 = -1.0 / 6.0
_S5 = 1.0 / 120.0
_S7 = -1.0 / 5040.0
_S9 = 1.0 / 362880.0


def _sin_reduced(ang):
    q = (ang * (1.0 / _PI)).astype(jnp.int32)
    r = ang - q.astype(jnp.float32) * _PI
    half = _PI / 2.0
    phi = half - jnp.abs(r - half)
    p2 = phi * phi
    s = phi * (1.0 + p2 * (_S3 + p2 * (_S5 + p2 * (_S7 + p2 * _S9))))
    return jnp.where((q & 1) == 0, s, -s)


def _tc_edge_kernel(d_ref, ev_ref, rbf_ref, fcut_ref, uvec_ref):
    d = d_ref[...].reshape(1, _BE)
    theta = d * (_PI / CUTOFF)
    inv_d = 1.0 / d
    w = _A * inv_d

    karr = (lax.broadcasted_iota(jnp.int32, (8, 1), 0) + 1).astype(jnp.float32)
    ang = karr * theta
    s8 = _sin_reduced(ang)
    c8 = _sin_reduced(ang + (_PI / 2.0))

    s8r = s8[7:8, :]
    c8r = c8[7:8, :]
    s16 = 2.0 * s8r * c8r
    c16 = 1.0 - 2.0 * s8r * s8r

    rbf_ref[0:8, :] = w * s8
    rbf_ref[8:16, :] = w * (s8r * c8 + c8r * s8)
    slab3 = w * (s16 * c8 + c16 * s8)
    rbf_ref[16:NUM_BASIS, :] = slab3[0 : NUM_BASIS - 16, :]

    c1 = c8[0:1, :]
    fcut_ref[...] = jnp.where(
        d < CUTOFF, 0.5 * (c1 + 1.0), 0.0
    ).reshape(1, 1, _BE)
    uvec_ref[...] = ev_ref[...] * inv_d


def _tc_edges(edge_vector, edge_length):
    d2 = edge_length.reshape(_GRID, 1, _BE)
    ev_t = edge_vector.T
    rbf_t, fcut2, uvec_t = pl.pallas_call(
        _tc_edge_kernel,
        grid=(_GRID,),
        in_specs=[
            pl.BlockSpec((1, 1, _BE), lambda i: (i, 0, 0)),
            pl.BlockSpec((3, _BE), lambda i: (0, i)),
        ],
        out_specs=[
            pl.BlockSpec((NUM_BASIS, _BE), lambda i: (0, i)),
            pl.BlockSpec((1, 1, _BE), lambda i: (i, 0, 0)),
            pl.BlockSpec((3, _BE), lambda i: (0, i)),
        ],
        out_shape=[
            jax.ShapeDtypeStruct((NUM_BASIS, E), jnp.float32),
            jax.ShapeDtypeStruct((_GRID, 1, _BE), jnp.float32),
            jax.ShapeDtypeStruct((3, E), jnp.float32),
        ],
    )(d2, ev_t)
    rbf = rbf_t.T
    fcut = fcut2.reshape(E, 1)
    uvec = uvec_t.T
    return rbf, fcut, uvec


def kernel(atomic_numbers, edge_vector, edge_length, embed_table):
    node_invariant, node_equivariant = _sc_gather(atomic_numbers, embed_table)
    rbf, fcut, uvec = _tc_edges(edge_vector, edge_length)
    return (node_invariant, rbf, fcut, uvec, node_equivariant)


# speedup vs baseline: 1.0179x; 1.0179x over previous
import functools
import math

import jax
import jax.numpy as jnp
from jax import lax
from jax.experimental import pallas as pl
from jax.experimental.pallas import tpu as pltpu
from jax.experimental.pallas import tpu_sc as plsc

N = 100000
E = 1600000
NODE_DIM = 128
NUM_BASIS = 20
CUTOFF = 5.0

_NC = 2
_NS = 16
_NW = _NC * _NS
_BPW = 3128
_GCHUNK = 384
_NGCHUNK = 9
_N_PAD = _BPW * _NW

def _sc_gather_kernel(idx_hbm, table_hbm, out_hbm,
                      idx_v, buf0, buf1, gsem, osem):
    wid = lax.axis_index("s") * _NC + lax.axis_index("c")
    base = wid * _BPW
    count = jnp.where(wid == _NW - 1, N - (_NW - 1) * _BPW, _BPW)
    pltpu.sync_copy(idx_hbm.at[pl.ds(base, _BPW)], idx_v)
    bufs = [buf0, buf1]
    starts = [jnp.minimum(i * _GCHUNK, count - _GCHUNK)
              for i in range(_NGCHUNK)]

    def gather_start(i):
        return pltpu.async_copy(
            table_hbm.at[idx_v.at[pl.ds(starts[i], _GCHUNK)]],
            bufs[i % 2], gsem,
        )

    g = {0: gather_start(0), 1: gather_start(1)}
    outs = {}
    for i in range(_NGCHUNK):
        g[i].wait()
        outs[i] = pltpu.async_copy(
            bufs[i % 2], out_hbm.at[pl.ds(base + starts[i], _GCHUNK)], osem
        )
        outs[i].wait()
        if i + 2 < _NGCHUNK:
            g[i + 2] = gather_start(i + 2)


def _sc_gather(atomic_numbers, embed_table):
    idx = jnp.pad(atomic_numbers.astype(jnp.int32), (0, _N_PAD - N))
    mesh = plsc.VectorSubcoreMesh(core_axis_name="c", subcore_axis_name="s")
    kern = functools.partial(
        pl.kernel,
        mesh=mesh,
        compiler_params=pltpu.CompilerParams(needs_layout_passes=False),
        out_type=jax.ShapeDtypeStruct((N, NODE_DIM), jnp.float32),
        scratch_types=[
            pltpu.VMEM((_BPW,), jnp.int32),
            pltpu.VMEM((_GCHUNK, NODE_DIM), jnp.float32),
            pltpu.VMEM((_GCHUNK, NODE_DIM), jnp.float32),
            pltpu.SemaphoreType.DMA,
            pltpu.SemaphoreType.DMA,
        ],
    )(_sc_gather_kernel)
    return kern(idx, embed_table)


_BE = 12800
_GRID = E // _BE

_PI = math.pi
_A = math.sqrt(2.0 / CUTOFF)

_S3 = -1.0 / 6.0
_S5 = 1.0 / 120.0
_S7 = -1.0 / 5040.0
_S9 = 1.0 / 362880.0


def _sin_reduced(ang):
    q = (ang * (1.0 / _PI)).astype(jnp.int32)
    r = ang - q.astype(jnp.float32) * _PI
    half = _PI / 2.0
    phi = half - jnp.abs(r - half)
    p2 = phi * phi
    s = phi * (1.0 + p2 * (_S3 + p2 * (_S5 + p2 * (_S7 + p2 * _S9))))
    return jnp.where((q & 1) == 0, s, -s)


def _tc_edge_kernel(d_ref, ev_ref, rbf_ref, fcut_ref, uvec_ref):
    d = d_ref[...].reshape(1, _BE)
    theta = d * (_PI / CUTOFF)
    inv_d = 1.0 / d
    w = _A * inv_d

    karr = (lax.broadcasted_iota(jnp.int32, (8, 1), 0) + 1).astype(jnp.float32)
    ang = karr * theta
    s8 = _sin_reduced(ang)
    c8 = _sin_reduced(ang + (_PI / 2.0))

    s8r = s8[7:8, :]
    c8r = c8[7:8, :]
    s16 = 2.0 * s8r * c8r
    c16 = 1.0 - 2.0 * s8r * s8r

    rbf_ref[0:8, :] = w * s8
    rbf_ref[8:16, :] = w * (s8r * c8 + c8r * s8)
    slab3 = w * (s16 * c8 + c16 * s8)
    rbf_ref[16:NUM_BASIS, :] = slab3[0 : NUM_BASIS - 16, :]

    c1 = c8[0:1, :]
    fcut_ref[...] = jnp.where(
        d < CUTOFF, 0.5 * (c1 + 1.0), 0.0
    ).reshape(1, 1, _BE)
    uvec_ref[...] = ev_ref[...] * inv_d


def _tc_edges(edge_vector, edge_length):
    d2 = edge_length.reshape(_GRID, 1, _BE)
    ev_t = edge_vector.T
    rbf_t, fcut2, uvec_t = pl.pallas_call(
        _tc_edge_kernel,
        grid=(_GRID,),
        in_specs=[
            pl.BlockSpec((1, 1, _BE), lambda i: (i, 0, 0)),
            pl.BlockSpec((3, _BE), lambda i: (0, i)),
        ],
        out_specs=[
            pl.BlockSpec((NUM_BASIS, _BE), lambda i: (0, i)),
            pl.BlockSpec((1, 1, _BE), lambda i: (i, 0, 0)),
            pl.BlockSpec((3, _BE), lambda i: (0, i)),
        ],
        out_shape=[
            jax.ShapeDtypeStruct((NUM_BASIS, E), jnp.float32),
            jax.ShapeDtypeStruct((_GRID, 1, _BE), jnp.float32),
            jax.ShapeDtypeStruct((3, E), jnp.float32),
        ],
    )(d2, ev_t)
    rbf = rbf_t.T
    fcut = fcut2.reshape(E, 1)
    uvec = uvec_t.T
    return rbf, fcut, uvec


def kernel(atomic_numbers, edge_vector, edge_length, embed_table):
    node_invariant = _sc_gather(atomic_numbers, embed_table)
    rbf, fcut, uvec = _tc_edges(edge_vector, edge_length)
    node_equivariant = jnp.zeros((N, 3, NODE_DIM), dtype=jnp.float32)
    return (node_invariant, rbf, fcut, uvec, node_equivariant)
